# baseline (speedup 1.0000x reference)
import functools

import jax
import jax.numpy as jnp
from jax import lax
from jax.experimental import pallas as pl
from jax.experimental.pallas import tpu as pltpu
from jax.experimental.pallas import tpu_sc as plsc

_E = 8
_TOP_K = 2
_N_GROUP = 4
_TOPK_GROUP = 2
_D = 1024
_I = 1408
_IS = 2816
_T = 2048
_BIS = 256
_NJ = 11
_NW = 32
_TW = _T // _NW
_BT = 256
_NBLK = 24
_XGB = _E * _T // _BT


def _splat(v, i):
    return v.at[jnp.full((16,), i, jnp.int32)].get(mode="promise_in_bounds")


def _cumsum16(v):
    lane = lax.iota(jnp.int32, 16)
    for k in (1, 2, 4, 8):
        g = v.at[jnp.maximum(lane - k, 0)].get(mode="promise_in_bounds")
        v = v + jnp.where(lane >= k, g, 0)
    return v

def _wid():
    return lax.axis_index("s") * 2 + lax.axis_index("c")


def _mesh():
    return plsc.VectorSubcoreMesh(core_axis_name="c", subcore_axis_name="s")


def _logits_body(x_ref, gw_ref, o_ref):
    o_ref[...] = lax.dot_general(gw_ref[...], x_ref[...],
                                 (((1,), (1,)), ((), ())),
                                 preferred_element_type=jnp.float32)


def _route_body(logits_hbm, eb_hbm, wt_hbm, cnt_hbm, lbuf, ebuf, wbuf, cvec):
    wid = _wid()
    pltpu.sync_copy(logits_hbm.at[wid], lbuf)
    lane = lax.iota(jnp.int32, 16)
    acc = [jnp.zeros((16,), jnp.int32) for _ in range(_E)]
    for c in range(_TW // 16):
        sl = pl.ds(c * 16, 16)
        s = [lbuf[e, sl] for e in range(_E)]
        m = s[0]
        for e in range(1, _E):
            m = jnp.maximum(m, s[e])
        ex = [jnp.exp(v - m) for v in s]
        den = ex[0]
        for e in range(1, _E):
            den = den + ex[e]
        p = [v / den for v in ex]
        G = [jnp.maximum(p[2 * g], p[2 * g + 1]) for g in range(_N_GROUP)]
        keep = []
        for g in range(_N_GROUP):
            r = jnp.zeros((16,), jnp.int32)
            for g2 in range(_N_GROUP):
                if g2 == g:
                    continue
                gt = (G[g2] >= G[g]) if g2 < g else (G[g2] > G[g])
                r = r + jnp.where(gt, 1, 0)
            keep.append(r < _TOPK_GROUP)
        mp = [jnp.where(keep[e // 2], p[e], 0.0) for e in range(_E)]
        rk = []
        for e in range(_E):
            r = jnp.zeros((16,), jnp.int32)
            for e2 in range(_E):
                if e2 == e:
                    continue
                gt = (mp[e2] >= mp[e]) if e2 < e else (mp[e2] > mp[e])
                r = r + jnp.where(gt, 1, 0)
            rk.append(r)
        w = [jnp.where(rk[e] < _TOP_K, mp[e], 0.0) for e in range(_E)]
        wsum = w[0]
        for e in range(1, _E):
            wsum = wsum + w[e]
        inv = 1.0 / (wsum + 1e-20)
        e0 = jnp.zeros((16,), jnp.int32)
        e1 = jnp.zeros((16,), jnp.int32)
        w0 = jnp.zeros((16,), jnp.float32)
        w1 = jnp.zeros((16,), jnp.float32)
        for e in range(_E):
            is0 = rk[e] == 0
            is1 = rk[e] == 1
            e0 = e0 + jnp.where(is0, e, 0)
            e1 = e1 + jnp.where(is1, e, 0)
            cw = w[e] * inv
            w0 = w0 + jnp.where(is0, cw, 0.0)
            w1 = w1 + jnp.where(is1, cw, 0.0)
            acc[e] = acc[e] + _splat(_cumsum16(jnp.where(rk[e] < _TOP_K, 1, 0)), 15)
        ebuf[0, sl] = e0
        ebuf[1, sl] = e1
        wbuf[0, sl] = w0
        wbuf[1, sl] = w1
    cv = jnp.zeros((16,), jnp.int32)
    for e in range(_E):
        cv = cv + jnp.where(lane == e, acc[e], 0)
    cvec[...] = cv
    pltpu.sync_copy(ebuf, eb_hbm.at[wid])
    pltpu.sync_copy(wbuf, wt_hbm.at[wid])
    pltpu.sync_copy(cvec, cnt_hbm.at[wid])


def _dispatch_body(x_hbm, eb_hbm, cnt_hbm, xg_hbm, pos_hbm, bd_hbm,
                   cnts, ebuf, posb, idx0, idx1, xrows, bdbuf, sem):
    wid = _wid()
    lane = lax.iota(jnp.int32, 16)
    pltpu.sync_copy(cnt_hbm, cnts)
    pltpu.sync_copy(eb_hbm.at[wid], ebuf)
    base = jnp.zeros((16,), jnp.int32)
    tot = jnp.zeros((16,), jnp.int32)
    for w2 in range(_NW):
        row = cnts[w2, :]
        tot = tot + row
        m = jnp.minimum(jnp.maximum(wid - w2, 0), 1)
        base = base + row * m
    carry = [_splat(base, e) for e in range(_E)]
    for c in range(_TW // 16):
        sl = pl.ds(c * 16, 16)
        for k in range(_TOP_K):
            av = ebuf[k, sl]
            posv = jnp.zeros((16,), jnp.int32)
            for e in range(_E):
                mask = av == e
                cums = _cumsum16(jnp.where(mask, 1, 0))
                posv = posv + jnp.where(mask, e * _T + carry[e] + cums - 1, 0)
                carry[e] = carry[e] + _splat(cums, 15)
            posb[k, sl] = posv
            if k == 0:
                idx0[sl] = posv
            else:
                idx1[sl] = posv
    pltpu.sync_copy(posb, pos_hbm.at[wid])
    pltpu.sync_copy(x_hbm.at[pl.ds(wid * _TW, _TW)], xrows)
    pltpu.async_copy(xrows, xg_hbm.at[idx0], sem).wait()
    pltpu.async_copy(xrows, xg_hbm.at[idx1], sem).wait()

    @pl.when(wid == 0)
    def _bdesc():
        nb = (tot + (_BT - 1)) >> 8
        incl = _cumsum16(nb)
        start = incl - nb
        tb = _splat(incl, _E - 1)
        for cb in range(2):
            slb = pl.ds(cb * 16, 16)
            bv = cb * 16 + lane
            valid = bv < tb
            bcl = jnp.minimum(bv, tb - 1)
            be = jnp.zeros((16,), jnp.int32)
            bx = jnp.zeros((16,), jnp.int32)
            for e in range(_E):
                se = _splat(start, e)
                ne = _splat(nb, e)
                ine = jnp.where(bcl >= se, 1, 0) * jnp.where(bcl < se + ne, 1, 0)
                be = be + ine * e
                bx = bx + ine * (e * (_T // _BT) + (bcl - se))
            bdbuf[0, slb] = be
            bdbuf[1, slb] = bx
            bdbuf[2, slb] = jnp.where(valid, 1, 0)
        pltpu.sync_copy(bdbuf, bd_hbm)


def _grouped_body(be_ref, bx_ref, bv_ref, xg_ref, wgu_ref, wd_ref, yg_ref):
    b = pl.program_id(0)

    @pl.when(bv_ref[b] > 0)
    def _():
        xb = xg_ref[0].astype(jnp.bfloat16)
        wg = wgu_ref[0, :_I].astype(jnp.bfloat16)
        wu = wgu_ref[0, _I:].astype(jnp.bfloat16)
        g = lax.dot_general(xb, wg, (((1,), (1,)), ((), ())),
                            preferred_element_type=jnp.float32)
        u = lax.dot_general(xb, wu, (((1,), (1,)), ((), ())),
                            preferred_element_type=jnp.float32)
        h = (g * jax.nn.sigmoid(g)) * u
        yg_ref[0] = lax.dot_general(h.astype(jnp.bfloat16),
                                    wd_ref[0].astype(jnp.bfloat16),
                                    (((1,), (1,)), ((), ())),
                                    preferred_element_type=jnp.float32)


def _shared_body(x_ref, sg_ref, su_ref, sd_ref, o_ref):
    j = pl.program_id(0)

    @pl.when(j == 0)
    def _init():
        o_ref[...] = jnp.zeros_like(o_ref)

    x = x_ref[...]
    g = lax.dot_general(x, sg_ref[...].astype(jnp.bfloat16),
                        (((1,), (1,)), ((), ())),
                        preferred_element_type=jnp.float32)
    u = lax.dot_general(x, su_ref[...].astype(jnp.bfloat16),
                        (((1,), (1,)), ((), ())),
                        preferred_element_type=jnp.float32)
    h = (g * jax.nn.sigmoid(g)) * u
    o_ref[...] += lax.dot_general(h.astype(jnp.bfloat16),
                                  sd_ref[...].astype(jnp.bfloat16),
                                  (((1,), (1,)), ((), ())),
                                  preferred_element_type=jnp.float32)


def _combine_body(yg_hbm, sh_hbm, pos_hbm, wt_hbm, out_hbm,
                  pbuf, wbuf, idxa, idxb, buf0, buf1, shbuf, sem):
    wid = _wid()
    pltpu.sync_copy(pos_hbm.at[wid], pbuf)
    pltpu.sync_copy(wt_hbm.at[wid], wbuf)
    for half in range(2):
        t0 = half * 32
        for q in range(2):
            idxa[pl.ds(q * 16, 16)] = pbuf[0, pl.ds(t0 + q * 16, 16)]
            idxb[pl.ds(q * 16, 16)] = pbuf[1, pl.ds(t0 + q * 16, 16)]
        pltpu.async_copy(yg_hbm.at[idxa], buf0, sem).wait()
        pltpu.async_copy(yg_hbm.at[idxb], buf1, sem).wait()
        pltpu.sync_copy(sh_hbm.at[pl.ds(wid * _TW + t0, 32)], shbuf)
        for grp in range(2):
            w0g = wbuf[0, pl.ds(t0 + grp * 16, 16)]
            w1g = wbuf[1, pl.ds(t0 + grp * 16, 16)]
            for t2 in range(16):
                t = grp * 16 + t2
                w0 = _splat(w0g, t2)
                w1 = _splat(w1g, t2)

                def col(jq, _):
                    for u in range(16):
                        slj = pl.ds(pl.multiple_of((jq * 16 + u) * 16, 16), 16)
                        shbuf[t, slj] = (shbuf[t, slj] + w0 * buf0[t, slj]
                                         + w1 * buf1[t, slj])
                    return 0

                lax.fori_loop(0, _D // 256, col, 0)
        pltpu.sync_copy(shbuf, out_hbm.at[pl.ds(wid * _TW + t0, 32)])


@jax.jit
def _run(x, gate_w, w_gate_up, w_down, shared_w_gate_up, shared_w_down):
    lt = pl.pallas_call(
        _logits_body,
        out_shape=jax.ShapeDtypeStruct((_E, _T), jnp.float32),
    )(x, gate_w)
    logits3 = lt.reshape(_E, _NW, _TW).transpose(1, 0, 2)

    eb, wt, cnt = pl.kernel(
        _route_body,
        out_type=[
            jax.ShapeDtypeStruct((_NW, _TOP_K, _TW), jnp.int32),
            jax.ShapeDtypeStruct((_NW, _TOP_K, _TW), jnp.float32),
            jax.ShapeDtypeStruct((_NW, 16), jnp.int32),
        ],
        mesh=_mesh(),
        scratch_types=[
            pltpu.VMEM((_E, _TW), jnp.float32),
            pltpu.VMEM((_TOP_K, _TW), jnp.int32),
            pltpu.VMEM((_TOP_K, _TW), jnp.float32),
            pltpu.VMEM((16,), jnp.int32),
        ],
    )(logits3)

    xg, pos, bd = pl.kernel(
        _dispatch_body,
        out_type=[
            jax.ShapeDtypeStruct((_E * _T, _D), jnp.float32),
            jax.ShapeDtypeStruct((_NW, _TOP_K, _TW), jnp.int32),
            jax.ShapeDtypeStruct((3, 32), jnp.int32),
        ],
        mesh=_mesh(),
        scratch_types=[
            pltpu.VMEM((_NW, 16), jnp.int32),
            pltpu.VMEM((_TOP_K, _TW), jnp.int32),
            pltpu.VMEM((_TOP_K, _TW), jnp.int32),
            pltpu.VMEM((_TW,), jnp.int32),
            pltpu.VMEM((_TW,), jnp.int32),
            pltpu.VMEM((_TW, _D), jnp.float32),
            pltpu.VMEM((3, 32), jnp.int32),
            pltpu.SemaphoreType.DMA,
        ],
    )(x, eb, cnt)

    yg = pl.pallas_call(
        _grouped_body,
        grid_spec=pltpu.PrefetchScalarGridSpec(
            num_scalar_prefetch=3,
            grid=(_NBLK,),
            in_specs=[
                pl.BlockSpec((1, _BT, _D),
                             lambda b, be, bx, bv: (bx[b], 0, 0)),
                pl.BlockSpec((1, 2 * _I, _D),
                             lambda b, be, bx, bv: (be[b], 0, 0)),
                pl.BlockSpec((1, _D, _I),
                             lambda b, be, bx, bv: (be[b], 0, 0)),
            ],
            out_specs=pl.BlockSpec((1, _BT, _D),
                                   lambda b, be, bx, bv: (bx[b], 0, 0)),
        ),
        out_shape=jax.ShapeDtypeStruct((_XGB, _BT, _D), jnp.float32),
        compiler_params=pltpu.CompilerParams(
            dimension_semantics=("arbitrary",),
            vmem_limit_bytes=120 * 1024 * 1024,
        ),
    )(bd[0], bd[1], bd[2], xg.reshape(_XGB, _BT, _D), w_gate_up, w_down)

    sh = pl.pallas_call(
        _shared_body,
        grid=(_NJ,),
        in_specs=[
            pl.BlockSpec((_T, _D), lambda j: (0, 0)),
            pl.BlockSpec((_BIS, _D), lambda j: (j, 0)),
            pl.BlockSpec((_BIS, _D), lambda j: (_NJ + j, 0)),
            pl.BlockSpec((_D, _BIS), lambda j: (0, j)),
        ],
        out_specs=pl.BlockSpec((_T, _D), lambda j: (0, 0)),
        out_shape=jax.ShapeDtypeStruct((_T, _D), jnp.float32),
        compiler_params=pltpu.CompilerParams(
            dimension_semantics=("arbitrary",),
            vmem_limit_bytes=120 * 1024 * 1024,
        ),
    )(x.astype(jnp.bfloat16), shared_w_gate_up, shared_w_gate_up, shared_w_down)

    out = pl.kernel(
        _combine_body,
        out_type=jax.ShapeDtypeStruct((_T, _D), jnp.float32),
        mesh=_mesh(),
        scratch_types=[
            pltpu.VMEM((_TOP_K, _TW), jnp.int32),
            pltpu.VMEM((_TOP_K, _TW), jnp.float32),
            pltpu.VMEM((32,), jnp.int32),
            pltpu.VMEM((32,), jnp.int32),
            pltpu.VMEM((32, _D), jnp.float32),
            pltpu.VMEM((32, _D), jnp.float32),
            pltpu.VMEM((32, _D), jnp.float32),
            pltpu.SemaphoreType.DMA,
        ],
    )(yg.reshape(_E * _T, _D), sh, pos, wt)
    return out


def kernel(hidden_states, gate_w, w_gate_up, w_down, shared_w_gate_up, shared_w_down):
    x = hidden_states.reshape(-1, _D)
    out = _run(x, gate_w, w_gate_up, w_down, shared_w_gate_up, shared_w_down)
    return out.reshape(hidden_states.shape)


# speedup vs baseline: 1.0369x; 1.0369x over previous
import functools

import jax
import jax.numpy as jnp
from jax import lax
from jax.experimental import pallas as pl
from jax.experimental.pallas import tpu as pltpu
from jax.experimental.pallas import tpu_sc as plsc

_E = 8
_TOP_K = 2
_N_GROUP = 4
_TOPK_GROUP = 2
_D = 1024
_I = 1408
_IS = 2816
_T = 2048
_BIS = 256
_NJ = 11
_NW = 32
_TW = _T // _NW
_BT = 256
_NBLK = 24
_XGB = _E * _T // _BT


def _splat(v, i):
    return v.at[jnp.full((16,), i, jnp.int32)].get(mode="promise_in_bounds")


def _cumsum16(v):
    lane = lax.iota(jnp.int32, 16)
    for k in (1, 2, 4, 8):
        g = v.at[jnp.maximum(lane - k, 0)].get(mode="promise_in_bounds")
        v = v + jnp.where(lane >= k, g, 0)
    return v

def _wid():
    return lax.axis_index("s") * 2 + lax.axis_index("c")


def _mesh():
    return plsc.VectorSubcoreMesh(core_axis_name="c", subcore_axis_name="s")


def _logits_body(x_ref, gw_ref, o_ref):
    o_ref[...] = lax.dot_general(gw_ref[...], x_ref[...],
                                 (((1,), (1,)), ((), ())),
                                 preferred_element_type=jnp.float32)


def _route_body(logits_hbm, eb_hbm, wt_hbm, cnt_hbm, lbuf, ebuf, wbuf, cvec):
    wid = _wid()
    pltpu.sync_copy(logits_hbm.at[wid], lbuf)
    lane = lax.iota(jnp.int32, 16)
    acc = [jnp.zeros((16,), jnp.int32) for _ in range(_E)]
    for c in range(_TW // 16):
        sl = pl.ds(c * 16, 16)
        s = [lbuf[e, sl] for e in range(_E)]
        m = s[0]
        for e in range(1, _E):
            m = jnp.maximum(m, s[e])
        ex = [jnp.exp(v - m) for v in s]
        den = ex[0]
        for e in range(1, _E):
            den = den + ex[e]
        p = [v / den for v in ex]
        G = [jnp.maximum(p[2 * g], p[2 * g + 1]) for g in range(_N_GROUP)]
        keep = []
        for g in range(_N_GROUP):
            r = jnp.zeros((16,), jnp.int32)
            for g2 in range(_N_GROUP):
                if g2 == g:
                    continue
                gt = (G[g2] >= G[g]) if g2 < g else (G[g2] > G[g])
                r = r + jnp.where(gt, 1, 0)
            keep.append(r < _TOPK_GROUP)
        mp = [jnp.where(keep[e // 2], p[e], 0.0) for e in range(_E)]
        rk = []
        for e in range(_E):
            r = jnp.zeros((16,), jnp.int32)
            for e2 in range(_E):
                if e2 == e:
                    continue
                gt = (mp[e2] >= mp[e]) if e2 < e else (mp[e2] > mp[e])
                r = r + jnp.where(gt, 1, 0)
            rk.append(r)
        w = [jnp.where(rk[e] < _TOP_K, mp[e], 0.0) for e in range(_E)]
        wsum = w[0]
        for e in range(1, _E):
            wsum = wsum + w[e]
        inv = 1.0 / (wsum + 1e-20)
        e0 = jnp.zeros((16,), jnp.int32)
        e1 = jnp.zeros((16,), jnp.int32)
        w0 = jnp.zeros((16,), jnp.float32)
        w1 = jnp.zeros((16,), jnp.float32)
        for e in range(_E):
            is0 = rk[e] == 0
            is1 = rk[e] == 1
            e0 = e0 + jnp.where(is0, e, 0)
            e1 = e1 + jnp.where(is1, e, 0)
            cw = w[e] * inv
            w0 = w0 + jnp.where(is0, cw, 0.0)
            w1 = w1 + jnp.where(is1, cw, 0.0)
            acc[e] = acc[e] + _splat(_cumsum16(jnp.where(rk[e] < _TOP_K, 1, 0)), 15)
        ebuf[0, sl] = e0
        ebuf[1, sl] = e1
        wbuf[0, sl] = w0
        wbuf[1, sl] = w1
    cv = jnp.zeros((16,), jnp.int32)
    for e in range(_E):
        cv = cv + jnp.where(lane == e, acc[e], 0)
    cvec[...] = cv
    pltpu.sync_copy(ebuf, eb_hbm.at[wid])
    pltpu.sync_copy(wbuf, wt_hbm.at[wid])
    pltpu.sync_copy(cvec, cnt_hbm.at[wid])


def _dispatch_body(x_hbm, eb_hbm, cnt_hbm, xg_hbm, pos_hbm, bd_hbm,
                   cnts, ebuf, posb, idx0, idx1, xrows, bdbuf, sem):
    wid = _wid()
    lane = lax.iota(jnp.int32, 16)
    pltpu.sync_copy(cnt_hbm, cnts)
    pltpu.sync_copy(eb_hbm.at[wid], ebuf)
    base = jnp.zeros((16,), jnp.int32)
    tot = jnp.zeros((16,), jnp.int32)
    for w2 in range(_NW):
        row = cnts[w2, :]
        tot = tot + row
        m = jnp.minimum(jnp.maximum(wid - w2, 0), 1)
        base = base + row * m
    carry = [_splat(base, e) for e in range(_E)]
    for c in range(_TW // 16):
        sl = pl.ds(c * 16, 16)
        for k in range(_TOP_K):
            av = ebuf[k, sl]
            posv = jnp.zeros((16,), jnp.int32)
            for e in range(_E):
                mask = av == e
                cums = _cumsum16(jnp.where(mask, 1, 0))
                posv = posv + jnp.where(mask, e * _T + carry[e] + cums - 1, 0)
                carry[e] = carry[e] + _splat(cums, 15)
            posb[k, sl] = posv
            if k == 0:
                idx0[sl] = posv
            else:
                idx1[sl] = posv
    pltpu.sync_copy(posb, pos_hbm.at[wid])
    pltpu.sync_copy(x_hbm.at[pl.ds(wid * _TW, _TW)], xrows)
    pltpu.async_copy(xrows, xg_hbm.at[idx0], sem).wait()
    pltpu.async_copy(xrows, xg_hbm.at[idx1], sem).wait()

    @pl.when(wid == 0)
    def _bdesc():
        nb = (tot + (_BT - 1)) >> 8
        incl = _cumsum16(nb)
        start = incl - nb
        tb = _splat(incl, _E - 1)
        for cb in range(2):
            slb = pl.ds(cb * 16, 16)
            bv = cb * 16 + lane
            valid = bv < tb
            bcl = jnp.minimum(bv, tb - 1)
            be = jnp.zeros((16,), jnp.int32)
            bx = jnp.zeros((16,), jnp.int32)
            for e in range(_E):
                se = _splat(start, e)
                ne = _splat(nb, e)
                ine = jnp.where(bcl >= se, 1, 0) * jnp.where(bcl < se + ne, 1, 0)
                be = be + ine * e
                bx = bx + ine * (e * (_T // _BT) + (bcl - se))
            bdbuf[0, slb] = be
            bdbuf[1, slb] = bx
            bdbuf[2, slb] = jnp.where(valid, 1, 0)
        pltpu.sync_copy(bdbuf, bd_hbm)


def _grouped_body(be_ref, bx_ref, bv_ref, xg_ref, wgu_ref, wd_ref, yg_ref):
    b = pl.program_id(0)

    @pl.when(bv_ref[b] > 0)
    def _():
        xb = xg_ref[0].astype(jnp.bfloat16)
        wg = wgu_ref[0, :_I].astype(jnp.bfloat16)
        wu = wgu_ref[0, _I:].astype(jnp.bfloat16)
        g = lax.dot_general(xb, wg, (((1,), (1,)), ((), ())),
                            preferred_element_type=jnp.float32)
        u = lax.dot_general(xb, wu, (((1,), (1,)), ((), ())),
                            preferred_element_type=jnp.float32)
        h = (g * jax.nn.sigmoid(g)) * u
        yg_ref[0] = lax.dot_general(h.astype(jnp.bfloat16),
                                    wd_ref[0].astype(jnp.bfloat16),
                                    (((1,), (1,)), ((), ())),
                                    preferred_element_type=jnp.float32)


def _shared_body(x_ref, sg_ref, su_ref, sd_ref, o_ref):
    j = pl.program_id(0)

    @pl.when(j == 0)
    def _init():
        o_ref[...] = jnp.zeros_like(o_ref)

    x = x_ref[...]
    g = lax.dot_general(x, sg_ref[...].astype(jnp.bfloat16),
                        (((1,), (1,)), ((), ())),
                        preferred_element_type=jnp.float32)
    u = lax.dot_general(x, su_ref[...].astype(jnp.bfloat16),
                        (((1,), (1,)), ((), ())),
                        preferred_element_type=jnp.float32)
    h = (g * jax.nn.sigmoid(g)) * u
    o_ref[...] += lax.dot_general(h.astype(jnp.bfloat16),
                                  sd_ref[...].astype(jnp.bfloat16),
                                  (((1,), (1,)), ((), ())),
                                  preferred_element_type=jnp.float32)


def _combine_body(yg_hbm, sh_hbm, pos_hbm, wt_hbm, out_hbm,
                  pbuf, wbuf, idxa, idxb, buf0, buf1, shbuf, sem):
    wid = _wid()
    pltpu.sync_copy(pos_hbm.at[wid], pbuf)
    pltpu.sync_copy(wt_hbm.at[wid], wbuf)
    for half in range(2):
        t0 = half * 32
        for q in range(2):
            idxa[pl.ds(q * 16, 16)] = pbuf[0, pl.ds(t0 + q * 16, 16)]
            idxb[pl.ds(q * 16, 16)] = pbuf[1, pl.ds(t0 + q * 16, 16)]
        pltpu.async_copy(yg_hbm.at[idxa], buf0, sem).wait()
        pltpu.async_copy(yg_hbm.at[idxb], buf1, sem).wait()
        pltpu.sync_copy(sh_hbm.at[pl.ds(wid * _TW + t0, 32)], shbuf)
        for grp in range(2):
            w0g = wbuf[0, pl.ds(t0 + grp * 16, 16)]
            w1g = wbuf[1, pl.ds(t0 + grp * 16, 16)]
            for t2 in range(16):
                t = grp * 16 + t2
                w0 = _splat(w0g, t2)
                w1 = _splat(w1g, t2)

                def col(jq, _):
                    for u in range(8):
                        slj = pl.ds(pl.multiple_of((jq * 8 + u) * 16, 16), 16)
                        shbuf[t, slj] = (shbuf[t, slj] + w0 * buf0[t, slj]
                                         + w1 * buf1[t, slj])
                    return 0

                lax.fori_loop(0, _D // 128, col, 0)
        pltpu.sync_copy(shbuf, out_hbm.at[pl.ds(wid * _TW + t0, 32)])


@jax.jit
def _run(x, gate_w, w_gate_up, w_down, shared_w_gate_up, shared_w_down):
    lt = pl.pallas_call(
        _logits_body,
        out_shape=jax.ShapeDtypeStruct((_E, _T), jnp.float32),
    )(x, gate_w)
    logits3 = lt.reshape(_E, _NW, _TW).transpose(1, 0, 2)

    eb, wt, cnt = pl.kernel(
        _route_body,
        out_type=[
            jax.ShapeDtypeStruct((_NW, _TOP_K, _TW), jnp.int32),
            jax.ShapeDtypeStruct((_NW, _TOP_K, _TW), jnp.float32),
            jax.ShapeDtypeStruct((_NW, 16), jnp.int32),
        ],
        mesh=_mesh(),
        scratch_types=[
            pltpu.VMEM((_E, _TW), jnp.float32),
            pltpu.VMEM((_TOP_K, _TW), jnp.int32),
            pltpu.VMEM((_TOP_K, _TW), jnp.float32),
            pltpu.VMEM((16,), jnp.int32),
        ],
    )(logits3)

    xg, pos, bd = pl.kernel(
        _dispatch_body,
        out_type=[
            jax.ShapeDtypeStruct((_E * _T, _D), jnp.float32),
            jax.ShapeDtypeStruct((_NW, _TOP_K, _TW), jnp.int32),
            jax.ShapeDtypeStruct((3, 32), jnp.int32),
        ],
        mesh=_mesh(),
        scratch_types=[
            pltpu.VMEM((_NW, 16), jnp.int32),
            pltpu.VMEM((_TOP_K, _TW), jnp.int32),
            pltpu.VMEM((_TOP_K, _TW), jnp.int32),
            pltpu.VMEM((_TW,), jnp.int32),
            pltpu.VMEM((_TW,), jnp.int32),
            pltpu.VMEM((_TW, _D), jnp.float32),
            pltpu.VMEM((3, 32), jnp.int32),
            pltpu.SemaphoreType.DMA,
        ],
    )(x, eb, cnt)

    yg = pl.pallas_call(
        _grouped_body,
        grid_spec=pltpu.PrefetchScalarGridSpec(
            num_scalar_prefetch=3,
            grid=(_NBLK,),
            in_specs=[
                pl.BlockSpec((1, _BT, _D),
                             lambda b, be, bx, bv: (bx[b], 0, 0)),
                pl.BlockSpec((1, 2 * _I, _D),
                             lambda b, be, bx, bv: (be[b], 0, 0)),
                pl.BlockSpec((1, _D, _I),
                             lambda b, be, bx, bv: (be[b], 0, 0)),
            ],
            out_specs=pl.BlockSpec((1, _BT, _D),
                                   lambda b, be, bx, bv: (bx[b], 0, 0)),
        ),
        out_shape=jax.ShapeDtypeStruct((_XGB, _BT, _D), jnp.float32),
        compiler_params=pltpu.CompilerParams(
            dimension_semantics=("arbitrary",),
            vmem_limit_bytes=120 * 1024 * 1024,
        ),
    )(bd[0], bd[1], bd[2], xg.reshape(_XGB, _BT, _D), w_gate_up, w_down)

    sh = pl.pallas_call(
        _shared_body,
        grid=(_NJ,),
        in_specs=[
            pl.BlockSpec((_T, _D), lambda j: (0, 0)),
            pl.BlockSpec((_BIS, _D), lambda j: (j, 0)),
            pl.BlockSpec((_BIS, _D), lambda j: (_NJ + j, 0)),
            pl.BlockSpec((_D, _BIS), lambda j: (0, j)),
        ],
        out_specs=pl.BlockSpec((_T, _D), lambda j: (0, 0)),
        out_shape=jax.ShapeDtypeStruct((_T, _D), jnp.float32),
        compiler_params=pltpu.CompilerParams(
            dimension_semantics=("arbitrary",),
            vmem_limit_bytes=120 * 1024 * 1024,
        ),
    )(x.astype(jnp.bfloat16), shared_w_gate_up, shared_w_gate_up, shared_w_down)

    out = pl.kernel(
        _combine_body,
        out_type=jax.ShapeDtypeStruct((_T, _D), jnp.float32),
        mesh=_mesh(),
        scratch_types=[
            pltpu.VMEM((_TOP_K, _TW), jnp.int32),
            pltpu.VMEM((_TOP_K, _TW), jnp.float32),
            pltpu.VMEM((32,), jnp.int32),
            pltpu.VMEM((32,), jnp.int32),
            pltpu.VMEM((32, _D), jnp.float32),
            pltpu.VMEM((32, _D), jnp.float32),
            pltpu.VMEM((32, _D), jnp.float32),
            pltpu.SemaphoreType.DMA,
        ],
    )(yg.reshape(_E * _T, _D), sh, pos, wt)
    return out


def kernel(hidden_states, gate_w, w_gate_up, w_down, shared_w_gate_up, shared_w_down):
    x = hidden_states.reshape(-1, _D)
    out = _run(x, gate_w, w_gate_up, w_down, shared_w_gate_up, shared_w_down)
    return out.reshape(hidden_states.shape)
